# baseline (speedup 1.0000x reference)
import functools

import jax
import jax.numpy as jnp
from jax import lax
from jax.experimental import pallas as pl
from jax.experimental.pallas import tpu as pltpu
from jax.experimental.pallas import tpu_sc as plsc

N_NODES = 100000
N_EDGES = 3200000
NC, NS, L = 2, 16, 16
NW = NC * NS
CH = 128
BLK_ROWS = 8
BLK = BLK_ROWS * CH
NBLK = N_EDGES // BLK
KMAX = -(-NBLK // NW)
PAIRS = (KMAX + 1) // 2
NPAD = 102400
ROWS128 = NPAD // 128
TSLICE = NPAD // NS

_SC_PARAMS = pltpu.CompilerParams(needs_layout_passes=False)
_MESH = plsc.VectorSubcoreMesh(core_axis_name="c", subcore_axis_name="s",
                               num_cores=NC, num_subcores=NS)


def _stage(ei_ref, b, dst, sem):
    return pltpu.async_copy(
        ei_ref.at[pl.ds(b * 2 * BLK_ROWS, 2 * BLK_ROWS)], dst, sem)


def _gather_pass_body(ei_ref, g_ref, zero_ref, out_ref, gtab,
                      ev0, co0, va0, ev1, co1, va1,
                      ev2, co2, va2, ev3, co3, va3,
                      bounce, acc,
                      sg0, sg1, sg2, sg3, ss0, ss1, ss2, ss3):
    c = lax.axis_index("c")
    s = lax.axis_index("s")
    wid = s * NC + c
    evs = (ev0, ev1, ev2, ev3)
    cos = (co0, co1, co2, co3)
    vas = (va0, va1, va2, va3)
    sgs = (sg0, sg1, sg2, sg3)
    sss = (ss0, ss1, ss2, ss3)

    pltpu.sync_copy(g_ref, gtab)
    pltpu.sync_copy(zero_ref.at[pl.ds(s * TSLICE, TSLICE)],
                    acc.at[pl.ds(s * TSLICE, TSLICE)])
    plsc.subcore_barrier()

    _stage(ei_ref, wid, ev0, sg0)
    _stage(ei_ref, wid + NW, ev1, sg1)

    def quad(k4, carry):
        for i in range(4):
            j4 = 4 * k4 + i
            b = wid + NW * j4
            q = (i + 2) % 4

            @pl.when(b < NBLK)
            def _(i=i, b=b):
                ev_v, cols_v, vals_v = evs[i], cos[i], vas[i]
                pltpu.make_async_copy(
                    ei_ref.at[pl.ds(b * 2 * BLK_ROWS, 2 * BLK_ROWS)],
                    ev_v, sgs[i]).wait()
                for j in range(BLK_ROWS):
                    for k in range(CH // L):
                        idx = ev_v[2 * j, pl.ds(k * L, L)]
                        vals_v[pl.ds(j * CH + k * L, L)] = (
                            plsc.load_gather(gtab, [idx]))
                for j in range(BLK_ROWS):
                    pltpu.async_copy(vals_v.at[pl.ds(j * CH, CH)],
                                     acc.at[ev_v.at[2 * j + 1]], sss[i],
                                     add=True)

            fired_jm2 = b - 2 * NW < NBLK
            if i < 2:
                fired_jm2 = jnp.logical_and(k4 > 0, fired_jm2)

            @pl.when(fired_jm2)
            def _(q=q):
                for j in range(BLK_ROWS):
                    pltpu.make_async_copy(vas[q].at[pl.ds(j * CH, CH)],
                                          acc.at[evs[q].at[2 * j + 1]],
                                          sss[q]).wait()

            @pl.when(b + 2 * NW < NBLK)
            def _(q=q, b=b):
                _stage(ei_ref, b + 2 * NW, evs[q], sgs[q])

        return carry

    lax.fori_loop(0, (KMAX + 2 + 3) // 4, quad, 0)
    plsc.subcore_barrier()

    for h in range(2):
        off = s * TSLICE + h * (TSLICE // 2)
        pltpu.sync_copy(acc.at[pl.ds(off, TSLICE // 2)], bounce)
        pltpu.sync_copy(bounce, out_ref.at[pl.ds(c * NPAD + off, TSLICE // 2)])


_gather_pass = pl.kernel(
    _gather_pass_body,
    out_type=jax.ShapeDtypeStruct((NC * NPAD,), jnp.float32),
    mesh=_MESH,
    scratch_types=(
        [pltpu.VMEM((NPAD,), jnp.float32)]
        + [pltpu.VMEM((2 * BLK_ROWS, CH), jnp.int32) if r == 0
           else pltpu.VMEM((BLK,), jnp.int32) if r == 1
           else pltpu.VMEM((BLK,), jnp.float32)
           for _ in range(4) for r in range(3)]
        + [pltpu.VMEM((TSLICE // 2,), jnp.float32),
           pltpu.VMEM_SHARED((NPAD,), jnp.float32)]
        + [pltpu.SemaphoreType.DMA] * 8
    ),
    compiler_params=_SC_PARAMS,
)


def _stage_cols(ei_ref, b, cols_v, sem):
    for j in range(BLK_ROWS):
        pltpu.async_copy(ei_ref.at[b * 2 * BLK_ROWS + 2 * j + 1],
                         cols_v.at[pl.ds(j * CH, CH)], sem)


def _wait_cols(ei_ref, b, cols_v, sem):
    for j in range(BLK_ROWS):
        pltpu.make_async_copy(ei_ref.at[b * 2 * BLK_ROWS + 2 * j + 1],
                              cols_v.at[pl.ds(j * CH, CH)], sem).wait()


def _deg_pass_body(ei_ref, zero_ref, out_ref,
                   co0, co1, co2, co3, ones_v, bounce, acc,
                   sg0, sg1, sg2, sg3, ss0, ss1, ss2, ss3):
    c = lax.axis_index("c")
    s = lax.axis_index("s")
    wid = s * NC + c
    cos = (co0, co1, co2, co3)
    sgs = (sg0, sg1, sg2, sg3)
    sss = (ss0, ss1, ss2, ss3)

    for i in range(BLK // L):
        ones_v[pl.ds(i * L, L)] = jnp.ones((L,), jnp.float32)
    pltpu.sync_copy(zero_ref.at[pl.ds(s * TSLICE, TSLICE)],
                    acc.at[pl.ds(s * TSLICE, TSLICE)])
    plsc.subcore_barrier()

    _stage_cols(ei_ref, wid, co0, sg0)
    _stage_cols(ei_ref, wid + NW, co1, sg1)

    def quad(k4, carry):
        for i in range(4):
            j4 = 4 * k4 + i
            b = wid + NW * j4
            q = (i + 2) % 4

            @pl.when(b < NBLK)
            def _(i=i, b=b):
                _wait_cols(ei_ref, b, cos[i], sgs[i])
                pltpu.async_copy(ones_v, acc.at[cos[i]], sss[i], add=True)

            fired_jm2 = b - 2 * NW < NBLK
            if i < 2:
                fired_jm2 = jnp.logical_and(k4 > 0, fired_jm2)

            @pl.when(fired_jm2)
            def _(q=q):
                pltpu.make_async_copy(ones_v, acc.at[cos[q]], sss[q]).wait()

            @pl.when(b + 2 * NW < NBLK)
            def _(q=q, b=b):
                _stage_cols(ei_ref, b + 2 * NW, cos[q], sgs[q])

        return carry

    lax.fori_loop(0, (KMAX + 2 + 3) // 4, quad, 0)
    plsc.subcore_barrier()

    pltpu.sync_copy(acc.at[pl.ds(s * TSLICE, TSLICE)], bounce)
    pltpu.sync_copy(bounce, out_ref.at[pl.ds(c * NPAD + s * TSLICE, TSLICE)])


_deg_pass = pl.kernel(
    _deg_pass_body,
    out_type=jax.ShapeDtypeStruct((NC * NPAD,), jnp.float32),
    mesh=_MESH,
    scratch_types=(
        [pltpu.VMEM((BLK,), jnp.int32)] * 4
        + [pltpu.VMEM((BLK,), jnp.float32),
           pltpu.VMEM((TSLICE,), jnp.float32),
           pltpu.VMEM_SHARED((NPAD,), jnp.float32)]
        + [pltpu.SemaphoreType.DMA] * 8
    ),
    compiler_params=_SC_PARAMS,
)


def _ew1_body(d_ref, x_ref, dis_ref, g_ref):
    deg = d_ref[0] + d_ref[1] + 1.0
    dis = lax.rsqrt(deg)
    dis_ref[...] = dis
    g_ref[...] = dis * x_ref[...]


def _ew2_body(p_ref, g_ref, dis_ref, w1_ref, b1_ref, w2_ref, g2_ref):
    dis = dis_ref[...]
    sv = dis * (p_ref[0] + p_ref[1] + g_ref[...])
    t = jnp.zeros_like(sv)
    for k in range(16):
        t = t + jnp.maximum(sv * w1_ref[0, k] + b1_ref[k], 0.0) * w2_ref[k, 0]
    g2_ref[...] = dis * t


def _ew3_body(p_ref, g2_ref, dis_ref, b2_ref, o_ref):
    u = dis_ref[...] * (p_ref[0] + p_ref[1] + g2_ref[...]) + b2_ref[0]
    o_ref[...] = 1.0 / (1.0 + jnp.exp(-u))


_V = functools.partial(pl.BlockSpec, memory_space=pltpu.MemorySpace.VMEM)
_S = functools.partial(pl.BlockSpec, memory_space=pltpu.MemorySpace.SMEM)
_F = jax.ShapeDtypeStruct((ROWS128, 128), jnp.float32)

_ew1 = pl.pallas_call(_ew1_body, out_shape=(_F, _F),
                      in_specs=[_V(), _V()], out_specs=(_V(), _V()))
_ew2 = pl.pallas_call(_ew2_body, out_shape=_F,
                      in_specs=[_V(), _V(), _V(), _S(), _S(), _S()],
                      out_specs=_V())
_ew3 = pl.pallas_call(_ew3_body, out_shape=_F,
                      in_specs=[_V(), _V(), _V(), _S()], out_specs=_V())


def kernel(x, edge_index, W1, b1, W2, b2):
    ei = (edge_index.astype(jnp.int32)
          .reshape(2, N_EDGES // CH, CH)
          .transpose(1, 0, 2)
          .reshape(2 * (N_EDGES // CH), CH))
    xp = jnp.pad(x.astype(jnp.float32), (0, NPAD - N_NODES))
    zeros = jnp.zeros((NPAD,), jnp.float32)

    deg_p = _deg_pass(ei, zeros)
    dis, g = _ew1(deg_p.reshape(NC, ROWS128, 128), xp.reshape(ROWS128, 128))

    s_p = _gather_pass(ei, g.reshape(NPAD), zeros)
    g2 = _ew2(s_p.reshape(NC, ROWS128, 128), g, dis, W1, b1, W2)

    u_p = _gather_pass(ei, g2.reshape(NPAD), zeros)
    out = _ew3(u_p.reshape(NC, ROWS128, 128), g2, dis, b2)

    return out.reshape(NPAD)[:N_NODES].reshape(N_NODES, 1)


# speedup vs baseline: 1.1090x; 1.1090x over previous
import functools

import jax
import jax.numpy as jnp
from jax import lax
from jax.experimental import pallas as pl
from jax.experimental.pallas import tpu as pltpu
from jax.experimental.pallas import tpu_sc as plsc

N_NODES = 100000
N_EDGES = 3200000
NC, NS, L = 2, 16, 16
NW = NC * NS
CH = 128
BLK_ROWS = 8
BLK = BLK_ROWS * CH
NBLK = N_EDGES // BLK
KMAX = -(-NBLK // NW)
PAIRS = (KMAX + 1) // 2
NPAD = 102400
ROWS128 = NPAD // 128
TSLICE = NPAD // NS

_SC_PARAMS = pltpu.CompilerParams(needs_layout_passes=False)
_MESH = plsc.VectorSubcoreMesh(core_axis_name="c", subcore_axis_name="s",
                               num_cores=NC, num_subcores=NS)


def _chunk_dmas(ei_ref, b, rows_v, cols_v, sem, make_only):
    mk = pltpu.make_async_copy if make_only else pltpu.async_copy
    cps = []
    for j in range(BLK_ROWS):
        off = 2 * CH * (BLK_ROWS * b + j)
        if rows_v is not None:
            cps.append(mk(ei_ref.at[pl.ds(off, CH)],
                          rows_v.at[pl.ds(j * CH, CH)], sem))
        cps.append(mk(ei_ref.at[pl.ds(off + CH, CH)],
                      cols_v.at[pl.ds(j * CH, CH)], sem))
    return cps


def _gather_pass_body(ei_ref, g_ref, zero_ref, out_ref, gtab,
                      ro0, co0, va0, ro1, co1, va1,
                      ro2, co2, va2, ro3, co3, va3,
                      bounce, acc,
                      sg0, sg1, sg2, sg3, ss0, ss1, ss2, ss3):
    c = lax.axis_index("c")
    s = lax.axis_index("s")
    wid = s * NC + c
    ros = (ro0, ro1, ro2, ro3)
    cos = (co0, co1, co2, co3)
    vas = (va0, va1, va2, va3)
    sgs = (sg0, sg1, sg2, sg3)
    sss = (ss0, ss1, ss2, ss3)

    pltpu.sync_copy(g_ref, gtab)
    pltpu.sync_copy(zero_ref.at[pl.ds(s * TSLICE, TSLICE)],
                    acc.at[pl.ds(s * TSLICE, TSLICE)])
    plsc.subcore_barrier()

    _chunk_dmas(ei_ref, wid, ro0, co0, sg0, False)
    _chunk_dmas(ei_ref, wid + NW, ro1, co1, sg1, False)

    def quad(k4, carry):
        for i in range(4):
            j4 = 4 * k4 + i
            b = wid + NW * j4
            q = (i + 2) % 4

            @pl.when(b < NBLK)
            def _(i=i, b=b):
                rows_v, vals_v = ros[i], vas[i]
                for cp in _chunk_dmas(ei_ref, b, rows_v, cos[i], sgs[i],
                                      True):
                    cp.wait()
                for j in range(BLK // (4 * L)):
                    idxs = [rows_v[pl.ds((4 * j + m) * L, L)]
                            for m in range(4)]
                    gs = [plsc.load_gather(gtab, [ix]) for ix in idxs]
                    for m in range(4):
                        vals_v[pl.ds((4 * j + m) * L, L)] = gs[m]
                pltpu.async_copy(vals_v, acc.at[cos[i]], sss[i], add=True)

            fired_jm2 = b - 2 * NW < NBLK
            if i < 2:
                fired_jm2 = jnp.logical_and(k4 > 0, fired_jm2)

            @pl.when(fired_jm2)
            def _(q=q):
                pltpu.make_async_copy(vas[q], acc.at[cos[q]], sss[q]).wait()

            @pl.when(b + 2 * NW < NBLK)
            def _(q=q, b=b):
                _chunk_dmas(ei_ref, b + 2 * NW, ros[q], cos[q], sgs[q], False)

        return carry

    lax.fori_loop(0, (KMAX + 2 + 3) // 4, quad, 0)
    plsc.subcore_barrier()

    for h in range(2):
        off = s * TSLICE + h * (TSLICE // 2)
        pltpu.sync_copy(acc.at[pl.ds(off, TSLICE // 2)], bounce)
        pltpu.sync_copy(bounce, out_ref.at[pl.ds(c * NPAD + off, TSLICE // 2)])


_gather_pass = pl.kernel(
    _gather_pass_body,
    out_type=jax.ShapeDtypeStruct((NC * NPAD,), jnp.float32),
    mesh=_MESH,
    scratch_types=(
        [pltpu.VMEM((NPAD,), jnp.float32)]
        + [pltpu.VMEM((BLK,), jnp.int32) if r < 2
           else pltpu.VMEM((BLK,), jnp.float32)
           for _ in range(4) for r in range(3)]
        + [pltpu.VMEM((TSLICE // 2,), jnp.float32),
           pltpu.VMEM_SHARED((NPAD,), jnp.float32)]
        + [pltpu.SemaphoreType.DMA] * 8
    ),
    compiler_params=_SC_PARAMS,
)


def _stage_cols(ei_ref, b, cols_v, sem):
    _chunk_dmas(ei_ref, b, None, cols_v, sem, False)


def _wait_cols(ei_ref, b, cols_v, sem):
    for cp in _chunk_dmas(ei_ref, b, None, cols_v, sem, True):
        cp.wait()


def _deg_pass_body(ei_ref, zero_ref, out_ref,
                   co0, co1, co2, co3, ones_v, bounce, acc,
                   sg0, sg1, sg2, sg3, ss0, ss1, ss2, ss3):
    c = lax.axis_index("c")
    s = lax.axis_index("s")
    wid = s * NC + c
    cos = (co0, co1, co2, co3)
    sgs = (sg0, sg1, sg2, sg3)
    sss = (ss0, ss1, ss2, ss3)

    for i in range(BLK // L):
        ones_v[pl.ds(i * L, L)] = jnp.ones((L,), jnp.float32)
    pltpu.sync_copy(zero_ref.at[pl.ds(s * TSLICE, TSLICE)],
                    acc.at[pl.ds(s * TSLICE, TSLICE)])
    plsc.subcore_barrier()

    _stage_cols(ei_ref, wid, co0, sg0)
    _stage_cols(ei_ref, wid + NW, co1, sg1)

    def quad(k4, carry):
        for i in range(4):
            j4 = 4 * k4 + i
            b = wid + NW * j4
            q = (i + 2) % 4

            @pl.when(b < NBLK)
            def _(i=i, b=b):
                _wait_cols(ei_ref, b, cos[i], sgs[i])
                pltpu.async_copy(ones_v, acc.at[cos[i]], sss[i], add=True)

            fired_jm2 = b - 2 * NW < NBLK
            if i < 2:
                fired_jm2 = jnp.logical_and(k4 > 0, fired_jm2)

            @pl.when(fired_jm2)
            def _(q=q):
                pltpu.make_async_copy(ones_v, acc.at[cos[q]], sss[q]).wait()

            @pl.when(b + 2 * NW < NBLK)
            def _(q=q, b=b):
                _stage_cols(ei_ref, b + 2 * NW, cos[q], sgs[q])

        return carry

    lax.fori_loop(0, (KMAX + 2 + 3) // 4, quad, 0)
    plsc.subcore_barrier()

    pltpu.sync_copy(acc.at[pl.ds(s * TSLICE, TSLICE)], bounce)
    pltpu.sync_copy(bounce, out_ref.at[pl.ds(c * NPAD + s * TSLICE, TSLICE)])


_deg_pass = pl.kernel(
    _deg_pass_body,
    out_type=jax.ShapeDtypeStruct((NC * NPAD,), jnp.float32),
    mesh=_MESH,
    scratch_types=(
        [pltpu.VMEM((BLK,), jnp.int32)] * 4
        + [pltpu.VMEM((BLK,), jnp.float32),
           pltpu.VMEM((TSLICE,), jnp.float32),
           pltpu.VMEM_SHARED((NPAD,), jnp.float32)]
        + [pltpu.SemaphoreType.DMA] * 8
    ),
    compiler_params=_SC_PARAMS,
)


def _ew1_body(d_ref, x_ref, dis_ref, g_ref):
    deg = d_ref[0] + d_ref[1] + 1.0
    dis = lax.rsqrt(deg)
    dis_ref[...] = dis
    g_ref[...] = dis * x_ref[...]


def _ew2_body(p_ref, g_ref, dis_ref, w1_ref, b1_ref, w2_ref, g2_ref):
    dis = dis_ref[...]
    sv = dis * (p_ref[0] + p_ref[1] + g_ref[...])
    t = jnp.zeros_like(sv)
    for k in range(16):
        t = t + jnp.maximum(sv * w1_ref[0, k] + b1_ref[k], 0.0) * w2_ref[k, 0]
    g2_ref[...] = dis * t


def _ew3_body(p_ref, g2_ref, dis_ref, b2_ref, o_ref):
    u = dis_ref[...] * (p_ref[0] + p_ref[1] + g2_ref[...]) + b2_ref[0]
    o_ref[...] = 1.0 / (1.0 + jnp.exp(-u))


_V = functools.partial(pl.BlockSpec, memory_space=pltpu.MemorySpace.VMEM)
_S = functools.partial(pl.BlockSpec, memory_space=pltpu.MemorySpace.SMEM)
_F = jax.ShapeDtypeStruct((ROWS128, 128), jnp.float32)

_ew1 = pl.pallas_call(_ew1_body, out_shape=(_F, _F),
                      in_specs=[_V(), _V()], out_specs=(_V(), _V()))
_ew2 = pl.pallas_call(_ew2_body, out_shape=_F,
                      in_specs=[_V(), _V(), _V(), _S(), _S(), _S()],
                      out_specs=_V())
_ew3 = pl.pallas_call(_ew3_body, out_shape=_F,
                      in_specs=[_V(), _V(), _V(), _S()], out_specs=_V())


def kernel(x, edge_index, W1, b1, W2, b2):
    ei = (edge_index.astype(jnp.int32)
          .reshape(2, N_EDGES // CH, CH)
          .transpose(1, 0, 2)
          .reshape(2 * N_EDGES))
    xp = jnp.pad(x.astype(jnp.float32), (0, NPAD - N_NODES))
    zeros = jnp.zeros((NPAD,), jnp.float32)

    deg_p = _deg_pass(ei, zeros)
    dis, g = _ew1(deg_p.reshape(NC, ROWS128, 128), xp.reshape(ROWS128, 128))

    s_p = _gather_pass(ei, g.reshape(NPAD), zeros)
    g2 = _ew2(s_p.reshape(NC, ROWS128, 128), g, dis, W1, b1, W2)

    u_p = _gather_pass(ei, g2.reshape(NPAD), zeros)
    out = _ew3(u_p.reshape(NC, ROWS128, 128), g2, dis, b2)

    return out.reshape(NPAD)[:N_NODES].reshape(N_NODES, 1)


# speedup vs baseline: 1.1256x; 1.0149x over previous
import functools

import jax
import jax.numpy as jnp
from jax import lax
from jax.experimental import pallas as pl
from jax.experimental.pallas import tpu as pltpu
from jax.experimental.pallas import tpu_sc as plsc

N_NODES = 100000
N_EDGES = 3200000
NC, NS, L = 2, 16, 16
NW = NC * NS
CH = 128
BLK_ROWS = 8
BLK = BLK_ROWS * CH
NBLK = N_EDGES // BLK
KMAX = -(-NBLK // NW)
PAIRS = (KMAX + 1) // 2
NPAD = 102400
ROWS128 = NPAD // 128
TSLICE = NPAD // NS

_SC_PARAMS = pltpu.CompilerParams(needs_layout_passes=False)
_MESH = plsc.VectorSubcoreMesh(core_axis_name="c", subcore_axis_name="s",
                               num_cores=NC, num_subcores=NS)


def _chunk_dmas(ei_ref, b, rows_v, cols_v, sem, make_only):
    mk = pltpu.make_async_copy if make_only else pltpu.async_copy
    cps = []
    for j in range(BLK_ROWS):
        off = 2 * CH * (BLK_ROWS * b + j)
        if rows_v is not None:
            cps.append(mk(ei_ref.at[pl.ds(off, CH)],
                          rows_v.at[pl.ds(j * CH, CH)], sem))
        cps.append(mk(ei_ref.at[pl.ds(off + CH, CH)],
                      cols_v.at[pl.ds(j * CH, CH)], sem))
    return cps


def _gather_pass_body(ei_ref, g_ref, zero_ref, out_ref, gtab,
                      ro0, co0, va0, ro1, co1, va1,
                      ro2, co2, va2, ro3, co3, va3,
                      bounce, acc,
                      sg0, sg1, sg2, sg3, ss0, ss1, ss2, ss3):
    c = lax.axis_index("c")
    s = lax.axis_index("s")
    wid = s * NC + c
    ros = (ro0, ro1, ro2, ro3)
    cos = (co0, co1, co2, co3)
    vas = (va0, va1, va2, va3)
    sgs = (sg0, sg1, sg2, sg3)
    sss = (ss0, ss1, ss2, ss3)

    tab_cp = pltpu.async_copy(g_ref, gtab, sg2)
    zero_cp = pltpu.async_copy(zero_ref.at[pl.ds(s * TSLICE, TSLICE)],
                               acc.at[pl.ds(s * TSLICE, TSLICE)], sg3)
    _chunk_dmas(ei_ref, wid, ro0, co0, sg0, False)
    _chunk_dmas(ei_ref, wid + NW, ro1, co1, sg1, False)
    tab_cp.wait()
    zero_cp.wait()
    plsc.subcore_barrier()

    def quad(k4, carry):
        for i in range(4):
            j4 = 4 * k4 + i
            b = wid + NW * j4
            q = (i + 2) % 4

            @pl.when(b < NBLK)
            def _(i=i, b=b):
                rows_v, vals_v = ros[i], vas[i]
                for cp in _chunk_dmas(ei_ref, b, rows_v, cos[i], sgs[i],
                                      True):
                    cp.wait()
                for j in range(BLK // (4 * L)):
                    idxs = [rows_v[pl.ds((4 * j + m) * L, L)]
                            for m in range(4)]
                    gs = [plsc.load_gather(gtab, [ix]) for ix in idxs]
                    for m in range(4):
                        vals_v[pl.ds((4 * j + m) * L, L)] = gs[m]
                pltpu.async_copy(vals_v, acc.at[cos[i]], sss[i], add=True)

            fired_jm2 = b - 2 * NW < NBLK
            if i < 2:
                fired_jm2 = jnp.logical_and(k4 > 0, fired_jm2)

            @pl.when(fired_jm2)
            def _(q=q):
                pltpu.make_async_copy(vas[q], acc.at[cos[q]], sss[q]).wait()

            @pl.when(b + 2 * NW < NBLK)
            def _(q=q, b=b):
                _chunk_dmas(ei_ref, b + 2 * NW, ros[q], cos[q], sgs[q], False)

        return carry

    lax.fori_loop(0, (KMAX + 2 + 3) // 4, quad, 0)
    plsc.subcore_barrier()

    for h in range(2):
        off = s * TSLICE + h * (TSLICE // 2)
        pltpu.sync_copy(acc.at[pl.ds(off, TSLICE // 2)], bounce)
        pltpu.sync_copy(bounce, out_ref.at[pl.ds(c * NPAD + off, TSLICE // 2)])


_gather_pass = pl.kernel(
    _gather_pass_body,
    out_type=jax.ShapeDtypeStruct((NC * NPAD,), jnp.float32),
    mesh=_MESH,
    scratch_types=(
        [pltpu.VMEM((NPAD,), jnp.float32)]
        + [pltpu.VMEM((BLK,), jnp.int32) if r < 2
           else pltpu.VMEM((BLK,), jnp.float32)
           for _ in range(4) for r in range(3)]
        + [pltpu.VMEM((TSLICE // 2,), jnp.float32),
           pltpu.VMEM_SHARED((NPAD,), jnp.float32)]
        + [pltpu.SemaphoreType.DMA] * 8
    ),
    compiler_params=_SC_PARAMS,
)


def _stage_cols(ei_ref, b, cols_v, sem):
    _chunk_dmas(ei_ref, b, None, cols_v, sem, False)


def _wait_cols(ei_ref, b, cols_v, sem):
    for cp in _chunk_dmas(ei_ref, b, None, cols_v, sem, True):
        cp.wait()


def _deg_pass_body(ei_ref, zero_ref, out_ref,
                   co0, co1, co2, co3, ones_v, bounce, acc,
                   sg0, sg1, sg2, sg3, ss0, ss1, ss2, ss3):
    c = lax.axis_index("c")
    s = lax.axis_index("s")
    wid = s * NC + c
    cos = (co0, co1, co2, co3)
    sgs = (sg0, sg1, sg2, sg3)
    sss = (ss0, ss1, ss2, ss3)

    zero_cp = pltpu.async_copy(zero_ref.at[pl.ds(s * TSLICE, TSLICE)],
                               acc.at[pl.ds(s * TSLICE, TSLICE)], sg2)
    _stage_cols(ei_ref, wid, co0, sg0)
    _stage_cols(ei_ref, wid + NW, co1, sg1)
    for i in range(BLK // L):
        ones_v[pl.ds(i * L, L)] = jnp.ones((L,), jnp.float32)
    zero_cp.wait()
    plsc.subcore_barrier()

    def quad(k4, carry):
        for i in range(4):
            j4 = 4 * k4 + i
            b = wid + NW * j4
            q = (i + 2) % 4

            @pl.when(b < NBLK)
            def _(i=i, b=b):
                _wait_cols(ei_ref, b, cos[i], sgs[i])
                pltpu.async_copy(ones_v, acc.at[cos[i]], sss[i], add=True)

            fired_jm2 = b - 2 * NW < NBLK
            if i < 2:
                fired_jm2 = jnp.logical_and(k4 > 0, fired_jm2)

            @pl.when(fired_jm2)
            def _(q=q):
                pltpu.make_async_copy(ones_v, acc.at[cos[q]], sss[q]).wait()

            @pl.when(b + 2 * NW < NBLK)
            def _(q=q, b=b):
                _stage_cols(ei_ref, b + 2 * NW, cos[q], sgs[q])

        return carry

    lax.fori_loop(0, (KMAX + 2 + 3) // 4, quad, 0)
    plsc.subcore_barrier()

    pltpu.sync_copy(acc.at[pl.ds(s * TSLICE, TSLICE)], bounce)
    pltpu.sync_copy(bounce, out_ref.at[pl.ds(c * NPAD + s * TSLICE, TSLICE)])


_deg_pass = pl.kernel(
    _deg_pass_body,
    out_type=jax.ShapeDtypeStruct((NC * NPAD,), jnp.float32),
    mesh=_MESH,
    scratch_types=(
        [pltpu.VMEM((BLK,), jnp.int32)] * 4
        + [pltpu.VMEM((BLK,), jnp.float32),
           pltpu.VMEM((TSLICE,), jnp.float32),
           pltpu.VMEM_SHARED((NPAD,), jnp.float32)]
        + [pltpu.SemaphoreType.DMA] * 8
    ),
    compiler_params=_SC_PARAMS,
)


def _ew1_body(d_ref, x_ref, dis_ref, g_ref):
    deg = d_ref[0] + d_ref[1] + 1.0
    dis = lax.rsqrt(deg)
    dis_ref[...] = dis
    g_ref[...] = dis * x_ref[...]


def _ew2_body(p_ref, g_ref, dis_ref, w1_ref, b1_ref, w2_ref, g2_ref):
    dis = dis_ref[...]
    sv = dis * (p_ref[0] + p_ref[1] + g_ref[...])
    t = jnp.zeros_like(sv)
    for k in range(16):
        t = t + jnp.maximum(sv * w1_ref[0, k] + b1_ref[k], 0.0) * w2_ref[k, 0]
    g2_ref[...] = dis * t


def _ew3_body(p_ref, g2_ref, dis_ref, b2_ref, o_ref):
    u = dis_ref[...] * (p_ref[0] + p_ref[1] + g2_ref[...]) + b2_ref[0]
    o_ref[...] = 1.0 / (1.0 + jnp.exp(-u))


_V = functools.partial(pl.BlockSpec, memory_space=pltpu.MemorySpace.VMEM)
_S = functools.partial(pl.BlockSpec, memory_space=pltpu.MemorySpace.SMEM)
_F = jax.ShapeDtypeStruct((ROWS128, 128), jnp.float32)

_ew1 = pl.pallas_call(_ew1_body, out_shape=(_F, _F),
                      in_specs=[_V(), _V()], out_specs=(_V(), _V()))
_ew2 = pl.pallas_call(_ew2_body, out_shape=_F,
                      in_specs=[_V(), _V(), _V(), _S(), _S(), _S()],
                      out_specs=_V())
_ew3 = pl.pallas_call(_ew3_body, out_shape=_F,
                      in_specs=[_V(), _V(), _V(), _S()], out_specs=_V())


def kernel(x, edge_index, W1, b1, W2, b2):
    ei = (edge_index.astype(jnp.int32)
          .reshape(2, N_EDGES // CH, CH)
          .transpose(1, 0, 2)
          .reshape(2 * N_EDGES))
    xp = jnp.pad(x.astype(jnp.float32), (0, NPAD - N_NODES))
    zeros = jnp.zeros((NPAD,), jnp.float32)

    deg_p = _deg_pass(ei, zeros)
    dis, g = _ew1(deg_p.reshape(NC, ROWS128, 128), xp.reshape(ROWS128, 128))

    s_p = _gather_pass(ei, g.reshape(NPAD), zeros)
    g2 = _ew2(s_p.reshape(NC, ROWS128, 128), g, dis, W1, b1, W2)

    u_p = _gather_pass(ei, g2.reshape(NPAD), zeros)
    out = _ew3(u_p.reshape(NC, ROWS128, 128), g2, dis, b2)

    return out.reshape(NPAD)[:N_NODES].reshape(N_NODES, 1)


# speedup vs baseline: 1.1285x; 1.0026x over previous
import functools

import jax
import jax.numpy as jnp
from jax import lax
from jax.experimental import pallas as pl
from jax.experimental.pallas import tpu as pltpu
from jax.experimental.pallas import tpu_sc as plsc

N_NODES = 100000
N_EDGES = 3200000
NC, NS, L = 2, 16, 16
NW = NC * NS
CH = 128
BLK_ROWS = 8
BLK = BLK_ROWS * CH
NBLK = N_EDGES // BLK
KMAX = -(-NBLK // NW)
NPAD = 102400
ROWS128 = NPAD // 128
TSLICE = NPAD // NS

_SC_PARAMS = pltpu.CompilerParams(needs_layout_passes=False)
_MESH = plsc.VectorSubcoreMesh(core_axis_name="c", subcore_axis_name="s",
                               num_cores=NC, num_subcores=NS)


def _chunk_dmas(ei_ref, b, rows_v, cols_v, sem, make_only):
    mk = pltpu.make_async_copy if make_only else pltpu.async_copy
    cps = []
    for j in range(BLK_ROWS):
        off = 2 * CH * (BLK_ROWS * b + j)
        if rows_v is not None:
            cps.append(mk(ei_ref.at[pl.ds(off, CH)],
                          rows_v.at[pl.ds(j * CH, CH)], sem))
        cps.append(mk(ei_ref.at[pl.ds(off + CH, CH)],
                      cols_v.at[pl.ds(j * CH, CH)], sem))
    return cps


def _gather_pass_body(ei_ref, g_ref, zero_ref, out_ref, gtab,
                      ro0, co0, va0, ro1, co1, va1,
                      ro2, co2, va2, ro3, co3, va3,
                      bounce, acc,
                      sg0, sg1, sg2, sg3, ss0, ss1, ss2, ss3):
    c = lax.axis_index("c")
    s = lax.axis_index("s")
    wid = s * NC + c
    ros = (ro0, ro1, ro2, ro3)
    cos = (co0, co1, co2, co3)
    vas = (va0, va1, va2, va3)
    sgs = (sg0, sg1, sg2, sg3)
    sss = (ss0, ss1, ss2, ss3)

    tab_cp = pltpu.async_copy(g_ref, gtab, sg2)
    zero_cp = pltpu.async_copy(zero_ref.at[pl.ds(s * TSLICE, TSLICE)],
                               acc.at[pl.ds(s * TSLICE, TSLICE)], sg3)
    _chunk_dmas(ei_ref, wid, ro0, co0, sg0, False)
    _chunk_dmas(ei_ref, wid + NW, ro1, co1, sg1, False)
    tab_cp.wait()
    zero_cp.wait()
    plsc.subcore_barrier()

    def quad(k4, carry):
        for i in range(4):
            j4 = 4 * k4 + i
            b = wid + NW * j4
            q = (i + 2) % 4

            @pl.when(b < NBLK)
            def _(i=i, b=b):
                rows_v, vals_v = ros[i], vas[i]
                for cp in _chunk_dmas(ei_ref, b, rows_v, cos[i], sgs[i],
                                      True):
                    cp.wait()
                for j in range(BLK // (4 * L)):
                    idxs = [rows_v[pl.ds((4 * j + m) * L, L)]
                            for m in range(4)]
                    gs = [plsc.load_gather(gtab, [ix]) for ix in idxs]
                    for m in range(4):
                        vals_v[pl.ds((4 * j + m) * L, L)] = gs[m]
                pltpu.async_copy(vals_v, acc.at[cos[i]], sss[i], add=True)

            fired_jm2 = b - 2 * NW < NBLK
            if i < 2:
                fired_jm2 = jnp.logical_and(k4 > 0, fired_jm2)

            @pl.when(fired_jm2)
            def _(q=q):
                pltpu.make_async_copy(vas[q], acc.at[cos[q]], sss[q]).wait()

            @pl.when(b + 2 * NW < NBLK)
            def _(q=q, b=b):
                _chunk_dmas(ei_ref, b + 2 * NW, ros[q], cos[q], sgs[q], False)

        return carry

    lax.fori_loop(0, (KMAX + 2 + 3) // 4, quad, 0)
    plsc.subcore_barrier()

    for h in range(2):
        off = s * TSLICE + h * (TSLICE // 2)
        pltpu.sync_copy(acc.at[pl.ds(off, TSLICE // 2)], bounce)
        pltpu.sync_copy(bounce, out_ref.at[pl.ds(c * NPAD + off, TSLICE // 2)])


_gather_pass = pl.kernel(
    _gather_pass_body,
    out_type=jax.ShapeDtypeStruct((NC * NPAD,), jnp.float32),
    mesh=_MESH,
    scratch_types=(
        [pltpu.VMEM((NPAD,), jnp.float32)]
        + [pltpu.VMEM((BLK,), jnp.int32) if r < 2
           else pltpu.VMEM((BLK,), jnp.float32)
           for _ in range(4) for r in range(3)]
        + [pltpu.VMEM((TSLICE // 2,), jnp.float32),
           pltpu.VMEM_SHARED((NPAD,), jnp.float32)]
        + [pltpu.SemaphoreType.DMA] * 8
    ),
    compiler_params=_SC_PARAMS,
)


def _stage_cols(ei_ref, b, cols_v, sem):
    _chunk_dmas(ei_ref, b, None, cols_v, sem, False)


def _wait_cols(ei_ref, b, cols_v, sem):
    for cp in _chunk_dmas(ei_ref, b, None, cols_v, sem, True):
        cp.wait()


def _deg_pass_body(ei_ref, zero_ref, out_ref,
                   co0, co1, co2, co3, ones_v, bounce, acc,
                   sg0, sg1, sg2, sg3, ss0, ss1, ss2, ss3):
    c = lax.axis_index("c")
    s = lax.axis_index("s")
    wid = s * NC + c
    cos = (co0, co1, co2, co3)
    sgs = (sg0, sg1, sg2, sg3)
    sss = (ss0, ss1, ss2, ss3)

    zero_cp = pltpu.async_copy(zero_ref.at[pl.ds(s * TSLICE, TSLICE)],
                               acc.at[pl.ds(s * TSLICE, TSLICE)], sg2)
    _stage_cols(ei_ref, wid, co0, sg0)
    _stage_cols(ei_ref, wid + NW, co1, sg1)
    for i in range(BLK // L):
        ones_v[pl.ds(i * L, L)] = jnp.ones((L,), jnp.float32)
    zero_cp.wait()
    plsc.subcore_barrier()

    def quad(k4, carry):
        for i in range(4):
            j4 = 4 * k4 + i
            b = wid + NW * j4
            q = (i + 2) % 4

            @pl.when(b < NBLK)
            def _(i=i, b=b):
                _wait_cols(ei_ref, b, cos[i], sgs[i])
                pltpu.async_copy(ones_v, acc.at[cos[i]], sss[i], add=True)

            fired_jm2 = b - 2 * NW < NBLK
            if i < 2:
                fired_jm2 = jnp.logical_and(k4 > 0, fired_jm2)

            @pl.when(fired_jm2)
            def _(q=q):
                pltpu.make_async_copy(ones_v, acc.at[cos[q]], sss[q]).wait()

            @pl.when(b + 2 * NW < NBLK)
            def _(q=q, b=b):
                _stage_cols(ei_ref, b + 2 * NW, cos[q], sgs[q])

        return carry

    lax.fori_loop(0, (KMAX + 2 + 3) // 4, quad, 0)
    plsc.subcore_barrier()

    pltpu.sync_copy(acc.at[pl.ds(s * TSLICE, TSLICE)], bounce)
    pltpu.sync_copy(bounce, out_ref.at[pl.ds(c * NPAD + s * TSLICE, TSLICE)])


_deg_pass = pl.kernel(
    _deg_pass_body,
    out_type=jax.ShapeDtypeStruct((NC * NPAD,), jnp.float32),
    mesh=_MESH,
    scratch_types=(
        [pltpu.VMEM((BLK,), jnp.int32)] * 4
        + [pltpu.VMEM((BLK,), jnp.float32),
           pltpu.VMEM((TSLICE,), jnp.float32),
           pltpu.VMEM_SHARED((NPAD,), jnp.float32)]
        + [pltpu.SemaphoreType.DMA] * 8
    ),
    compiler_params=_SC_PARAMS,
)


def _ew1_body(d_ref, x_ref, dis_ref, g_ref):
    deg = d_ref[0] + d_ref[1] + 1.0
    dis = lax.rsqrt(deg)
    dis_ref[...] = dis
    g_ref[...] = dis * x_ref[...]


def _ew2_body(p_ref, g_ref, dis_ref, w1_ref, b1_ref, w2_ref, g2_ref):
    dis = dis_ref[...]
    sv = dis * (p_ref[0] + p_ref[1] + g_ref[...])
    t = jnp.zeros_like(sv)
    for k in range(16):
        t = t + jnp.maximum(sv * w1_ref[0, k] + b1_ref[k], 0.0) * w2_ref[k, 0]
    g2_ref[...] = dis * t


def _ew3_body(p_ref, g2_ref, dis_ref, b2_ref, o_ref):
    u = dis_ref[...] * (p_ref[0] + p_ref[1] + g2_ref[...]) + b2_ref[0]
    o_ref[...] = 1.0 / (1.0 + jnp.exp(-u))


_V = functools.partial(pl.BlockSpec, memory_space=pltpu.MemorySpace.VMEM)
_S = functools.partial(pl.BlockSpec, memory_space=pltpu.MemorySpace.SMEM)
_F = jax.ShapeDtypeStruct((ROWS128, 128), jnp.float32)

_ew1 = pl.pallas_call(_ew1_body, out_shape=(_F, _F),
                      in_specs=[_V(), _V()], out_specs=(_V(), _V()))
_ew2 = pl.pallas_call(_ew2_body, out_shape=_F,
                      in_specs=[_V(), _V(), _V(), _S(), _S(), _S()],
                      out_specs=_V())
_ew3 = pl.pallas_call(_ew3_body, out_shape=_F,
                      in_specs=[_V(), _V(), _V(), _S()], out_specs=_V())


def kernel(x, edge_index, W1, b1, W2, b2):
    ei = (edge_index.astype(jnp.int32)
          .reshape(2, N_EDGES // CH, CH)
          .transpose(1, 0, 2)
          .reshape(2 * N_EDGES))
    xp = jnp.pad(x.astype(jnp.float32), (0, NPAD - N_NODES))
    zeros = jnp.zeros((NPAD,), jnp.float32)

    deg_p = _deg_pass(ei, zeros)
    dis, g = _ew1(deg_p.reshape(NC, ROWS128, 128), xp.reshape(ROWS128, 128))

    s_p = _gather_pass(ei, g.reshape(NPAD), zeros)
    g2 = _ew2(s_p.reshape(NC, ROWS128, 128), g, dis, W1, b1, W2)

    u_p = _gather_pass(ei, g2.reshape(NPAD), zeros)
    out = _ew3(u_p.reshape(NC, ROWS128, 128), g2, dis, b2)

    return out.reshape(NPAD)[:N_NODES].reshape(N_NODES, 1)
